# baseline (speedup 1.0000x reference)
import functools

import jax
import jax.numpy as jnp
from jax import lax
from jax.experimental import pallas as pl
from jax.experimental.pallas import tpu as pltpu
from jax.experimental.pallas import tpu_sc as plsc

NC = 2
NS = 16
NW = NC * NS
CH = 128


@functools.lru_cache(maxsize=None)
def _build(B, D):
    b_per_w = B // NW
    n_ch = b_per_w // CH
    mesh = plsc.VectorSubcoreMesh(core_axis_name="c", subcore_axis_name="s")

    @functools.partial(
        pl.kernel,
        mesh=mesh,
        out_type=(
            jax.ShapeDtypeStruct((B // CH, CH, D), jnp.float32),
            jax.ShapeDtypeStruct((B // CH, CH, D), jnp.float32),
        ),
        scratch_types=[
            pltpu.VMEM((n_ch, CH), jnp.int32),
            pltpu.VMEM((n_ch, CH), jnp.int32),
            pltpu.VMEM((n_ch, CH, D), jnp.float32),
            pltpu.VMEM((n_ch, CH, D), jnp.float32),
            pltpu.SemaphoreType.DMA,
            pltpu.SemaphoreType.DMA,
        ],
        compiler_params=pltpu.CompilerParams(use_tc_tiling_on_sc=False),
    )
    def k(ori_hbm, dest_hbm, table_hbm, out_o_hbm, out_d_hbm,
          idx_o, idx_d, rows_o, rows_d, sem_o, sem_d):
        wid = lax.axis_index("s") * NC + lax.axis_index("c")
        base = wid * n_ch
        pltpu.sync_copy(ori_hbm.at[pl.ds(base, n_ch)], idx_o)
        pltpu.sync_copy(dest_hbm.at[pl.ds(base, n_ch)], idx_d)
        copies_o = [
            pltpu.async_copy(table_hbm.at[idx_o.at[j]], rows_o.at[j], sem_o)
            for j in range(n_ch)
        ]
        copies_d = [
            pltpu.async_copy(table_hbm.at[idx_d.at[j]], rows_d.at[j], sem_d)
            for j in range(n_ch)
        ]
        for c in copies_o:
            c.wait()
        pltpu.sync_copy(rows_o, out_o_hbm.at[pl.ds(base, n_ch)])
        for c in copies_d:
            c.wait()
        pltpu.sync_copy(rows_d, out_d_hbm.at[pl.ds(base, n_ch)])

    return k


def kernel(ori, dest, table):
    B = ori.shape[0]
    D = table.shape[1]
    ori2 = ori.astype(jnp.int32).reshape(B // CH, CH)
    dest2 = dest.astype(jnp.int32).reshape(B // CH, CH)
    out_o, out_d = _build(B, D)(ori2, dest2, table)
    return out_o.reshape(B, D), out_d.reshape(B, D)
